# baseline (speedup 1.0000x reference)
import numpy as np
import jax
import jax.numpy as jnp
from jax.experimental import pallas as pl
from jax.experimental.pallas import tpu as pltpu

_MASK_ID = 32000
_TIMESTEPS = 10
_B, _S = 4, 8192
_R, _C = 32, 1024

_P_F32 = np.arange(_TIMESTEPS, dtype=np.float64) * 0.1
_THR = np.ceil(_P_F32.astype(np.float32).astype(np.float64) * 2.0**23).astype(
    np.int32
)

_KS = (np.uint32(0), np.uint32(42), np.uint32(0x1BD11BDA ^ 42))
_ROT = ((13, 15, 26, 6), (17, 29, 16, 24))


def _mask_kernel(t_ref, tokens_ref, noisy_ref, labels_ref, mask_ref):
    rows = jax.lax.broadcasted_iota(jnp.uint32, (_R, _C), 0)
    cols = jax.lax.broadcasted_iota(jnp.uint32, (_R, _C), 1)
    n = rows * jnp.uint32(_C) + cols

    x0 = jnp.full((_R, _C), _KS[0], jnp.uint32)
    x1 = n + _KS[1]
    for i in range(5):
        for r in _ROT[i % 2]:
            x0 = x0 + x1
            x1 = ((x1 << r) | (x1 >> (32 - r))) ^ x0
        x0 = x0 + _KS[(i + 1) % 3]
        x1 = x1 + _KS[(i + 2) % 3] + jnp.uint32(i + 1)
    mant = ((x0 ^ x1) >> 9).astype(jnp.int32)

    batch_row = jax.lax.broadcasted_iota(jnp.int32, (_R, 1), 0) >> 3
    thr = jnp.zeros((_R, 1), jnp.int32)
    for i in range(_B):
        ti = jnp.clip(t_ref[i], 0, _TIMESTEPS - 1)
        thr_i = jnp.int32(_THR[_TIMESTEPS - 1])
        for k in range(_TIMESTEPS - 1):
            thr_i = jnp.where(ti == k, jnp.int32(_THR[k]), thr_i)
        thr = jnp.where(batch_row == i, thr_i, thr)

    mask = mant < thr
    tokens = tokens_ref[...]
    noisy_ref[...] = jnp.where(mask, jnp.int32(_MASK_ID), tokens)
    labels_ref[...] = jnp.where(mask, tokens, jnp.int32(-100))
    mask_ref[...] = mask


def kernel(tokens, t):
    tok = tokens.reshape(_R, _C)
    noisy, labels, mask = pl.pallas_call(
        _mask_kernel,
        in_specs=[
            pl.BlockSpec(memory_space=pltpu.SMEM),
            pl.BlockSpec(memory_space=pltpu.VMEM),
        ],
        out_shape=(
            jax.ShapeDtypeStruct((_R, _C), jnp.int32),
            jax.ShapeDtypeStruct((_R, _C), jnp.int32),
            jax.ShapeDtypeStruct((_R, _C), jnp.bool_),
        ),
    )(t, tok)
    return (
        noisy.reshape(_B, _S),
        labels.reshape(_B, _S),
        t,
        mask.reshape(_B, _S),
    )


# speedup vs baseline: 1.4794x; 1.4794x over previous
import numpy as np
import jax
import jax.numpy as jnp
from jax.experimental import pallas as pl
from jax.experimental.pallas import tpu as pltpu

_MASK_ID = 32000
_TIMESTEPS = 10
_B, _S = 4, 8192

_P_F32 = np.arange(_TIMESTEPS, dtype=np.float64) * 0.1
_THR = np.ceil(_P_F32.astype(np.float32).astype(np.float64) * 2.0**23).astype(
    np.int32
)

_KS = (np.uint32(0), np.uint32(42), np.uint32(0x1BD11BDA ^ 42))
_ROT = ((13, 15, 26, 6), (17, 29, 16, 24))


def _mask_kernel(t_ref, tokens_ref, noisy_ref, labels_ref, mask_ref):
    rows = jax.lax.broadcasted_iota(jnp.uint32, (_B, _S), 0)
    cols = jax.lax.broadcasted_iota(jnp.uint32, (_B, _S), 1)
    n = rows * jnp.uint32(_S) + cols

    x0 = jnp.full((_B, _S), _KS[0], jnp.uint32)
    x1 = n + _KS[1]
    for i in range(5):
        for r in _ROT[i % 2]:
            x0 = x0 + x1
            x1 = ((x1 << r) | (x1 >> (32 - r))) ^ x0
        x0 = x0 + _KS[(i + 1) % 3]
        x1 = x1 + _KS[(i + 2) % 3] + jnp.uint32(i + 1)
    mant = ((x0 ^ x1) >> 9).astype(jnp.int32)

    batch_row = jax.lax.broadcasted_iota(jnp.int32, (_B, 1), 0)
    thr = jnp.zeros((_B, 1), jnp.int32)
    for i in range(_B):
        ti = jnp.clip(t_ref[i], 0, _TIMESTEPS - 1)
        thr_i = jnp.int32(_THR[_TIMESTEPS - 1])
        for k in range(_TIMESTEPS - 1):
            thr_i = jnp.where(ti == k, jnp.int32(_THR[k]), thr_i)
        thr = jnp.where(batch_row == i, thr_i, thr)

    mask = mant < thr
    tokens = tokens_ref[...]
    noisy_ref[...] = jnp.where(mask, jnp.int32(_MASK_ID), tokens)
    labels_ref[...] = jnp.where(mask, tokens, jnp.int32(-100))
    mask_ref[...] = mask


def kernel(tokens, t):
    noisy, labels, mask = pl.pallas_call(
        _mask_kernel,
        in_specs=[
            pl.BlockSpec(memory_space=pltpu.SMEM),
            pl.BlockSpec(memory_space=pltpu.VMEM),
        ],
        out_shape=(
            jax.ShapeDtypeStruct((_B, _S), jnp.int32),
            jax.ShapeDtypeStruct((_B, _S), jnp.int32),
            jax.ShapeDtypeStruct((_B, _S), jnp.bool_),
        ),
    )(t, tokens)
    return (noisy, labels, t, mask)
